# baseline (speedup 1.0000x reference)
import jax
import jax.numpy as jnp
from jax import lax
from jax.experimental import pallas as pl
from jax.experimental.pallas import tpu as pltpu
from jax.experimental.pallas import tpu_sc as plsc

NC = 2
NS = 16
NW = NC * NS
CHUNK = 100
DP = 64


def _sc_degree(nchunk, nacc):
    slab = nacc // NS

    def body(dst_hbm, ones_hbm, zer_hbm, out_hbm, didx, ones_v, acc):
        cid = lax.axis_index("c")
        sid = lax.axis_index("s")
        wid = cid * NS + sid
        pltpu.sync_copy(zer_hbm, acc.at[pl.ds(sid * slab, slab)])
        pltpu.sync_copy(dst_hbm.at[wid], didx)
        pltpu.sync_copy(ones_hbm, ones_v)
        plsc.subcore_barrier()

        def step(j, carry):
            pltpu.sync_copy(ones_v, acc.at[didx.at[j]], add=True)
            return carry

        lax.fori_loop(0, nchunk, step, 0)
        plsc.subcore_barrier()
        pltpu.sync_copy(acc.at[pl.ds(sid * slab, slab)],
                        out_hbm.at[pl.ds(cid * nacc + sid * slab, slab)])

    return pl.kernel(
        body,
        out_type=jax.ShapeDtypeStruct((NC * nacc,), jnp.float32),
        mesh=plsc.VectorSubcoreMesh(core_axis_name="c", subcore_axis_name="s"),
        compiler_params=pltpu.CompilerParams(use_tc_tiling_on_sc=False),
        scratch_types=[
            pltpu.VMEM((nchunk, CHUNK), jnp.int32),
            pltpu.VMEM((CHUNK,), jnp.float32),
            pltpu.VMEM_SHARED((nacc,), jnp.float32),
        ],
    )


def _sc_messages(nchunk, nacc, n, npass):
    slab = nacc // NS
    stripe = n // NS

    def body(src_hbm, dst_hbm, *refs):
        tabs_hbm = refs[:npass]
        zer_hbm = refs[npass]
        outs_hbm = refs[npass + 1:2 * npass + 1]
        rest = refs[2 * npass + 1:]
        sidx, didx = rest[0], rest[1]
        rows = rest[2:6]
        tab, acc = rest[6], rest[7]
        gsem = rest[8:12]
        ssem = rest[12:16]
        cid = lax.axis_index("c")
        sid = lax.axis_index("s")
        wid = cid * NS + sid
        pltpu.sync_copy(src_hbm.at[wid], sidx)
        pltpu.sync_copy(dst_hbm.at[wid], didx)

        for p in range(npass):
            pltpu.sync_copy(zer_hbm, acc.at[pl.ds(sid * slab, slab)])
            pltpu.sync_copy(tabs_hbm[p].at[pl.ds(sid * stripe, stripe)],
                            tab.at[pl.ds(sid * stripe, stripe)])
            plsc.subcore_barrier()

            for b in range(3):
                pltpu.async_copy(tab.at[sidx.at[b]], rows[b], gsem[b])

            def phase(j, b, first):
                pltpu.make_async_copy(tab.at[sidx.at[j]], rows[b],
                                      gsem[b]).wait()
                pltpu.async_copy(rows[b], acc.at[didx.at[j]], ssem[b],
                                 add=True)
                nb = (b + 3) % 4
                if first:
                    pltpu.async_copy(tab.at[sidx.at[j + 3]], rows[nb],
                                     gsem[nb])
                else:
                    @pl.when(j + 3 < nchunk)
                    def _():
                        pltpu.make_async_copy(rows[nb],
                                              acc.at[didx.at[j - 1]],
                                              ssem[nb]).wait()
                        pltpu.async_copy(tab.at[sidx.at[j + 3]], rows[nb],
                                         gsem[nb])

            phase(0, 0, True)
            phase(1, 1, False)
            phase(2, 2, False)
            phase(3, 3, False)

            def step(i, carry):
                for b in range(4):
                    phase(4 * i + b, b, False)
                return carry

            lax.fori_loop(1, nchunk // 4, step, 0)
            for b in range(4):
                pltpu.make_async_copy(rows[b], acc.at[didx.at[0]],
                                      ssem[b]).wait()
            plsc.subcore_barrier()
            pltpu.sync_copy(acc.at[pl.ds(sid * slab, slab)],
                            outs_hbm[p].at[cid, pl.ds(sid * slab, slab)])
            if p + 1 < npass:
                plsc.subcore_barrier()

    return pl.kernel(
        body,
        out_type=[jax.ShapeDtypeStruct((NC, nacc, DP), jnp.float32)
                  for _ in range(npass)],
        mesh=plsc.VectorSubcoreMesh(core_axis_name="c", subcore_axis_name="s"),
        compiler_params=pltpu.CompilerParams(use_tc_tiling_on_sc=False),
        scratch_types=[
            pltpu.VMEM((nchunk, CHUNK), jnp.int32),
            pltpu.VMEM((nchunk, CHUNK), jnp.int32),
            pltpu.VMEM((CHUNK, DP), jnp.float32),
            pltpu.VMEM((CHUNK, DP), jnp.float32),
            pltpu.VMEM((CHUNK, DP), jnp.float32),
            pltpu.VMEM((CHUNK, DP), jnp.float32),
            pltpu.VMEM_SHARED((nacc, DP), jnp.float32),
            pltpu.VMEM_SHARED((nacc, DP), jnp.float32),
            pltpu.SemaphoreType.DMA,
            pltpu.SemaphoreType.DMA,
            pltpu.SemaphoreType.DMA,
            pltpu.SemaphoreType.DMA,
            pltpu.SemaphoreType.DMA,
            pltpu.SemaphoreType.DMA,
            pltpu.SemaphoreType.DMA,
            pltpu.SemaphoreType.DMA,
        ],
    )


def _tc_matmul(n, bm, d_in, d_hid):
    def body(x_ref, w_ref, o_ref):
        o_ref[...] = jnp.dot(x_ref[...], w_ref[...],
                             preferred_element_type=jnp.float32)

    return pl.pallas_call(
        body,
        grid=(n // bm,),
        in_specs=[
            pl.BlockSpec((bm, d_in), lambda i: (i, 0)),
            pl.BlockSpec((d_in, d_hid), lambda i: (0, 0)),
        ],
        out_specs=pl.BlockSpec((bm, d_hid), lambda i: (i, 0)),
        out_shape=jax.ShapeDtypeStruct((n, d_hid), jnp.float32),
    )


def _tc_scale_split(n, bm, d_hid):
    def body(h_ref, dis_ref, oa_ref, ob_ref):
        hs = h_ref[...] * dis_ref[...]
        oa_ref[...] = hs[:, :DP]
        ob_ref[...] = hs[:, DP:]

    return pl.pallas_call(
        body,
        grid=(n // bm,),
        in_specs=[
            pl.BlockSpec((bm, d_hid), lambda i: (i, 0)),
            pl.BlockSpec((bm, 1), lambda i: (i, 0)),
        ],
        out_specs=[
            pl.BlockSpec((bm, DP), lambda i: (i, 0)),
            pl.BlockSpec((bm, DP), lambda i: (i, 0)),
        ],
        out_shape=[
            jax.ShapeDtypeStruct((n, DP), jnp.float32),
            jax.ShapeDtypeStruct((n, DP), jnp.float32),
        ],
    )


def _tc_layer2(n, bm, d_hid, d_out):
    def body(aa_ref, ab_ref, hsa_ref, hsb_ref, dis_ref, b_ref, w_ref, o_ref):
        dis = dis_ref[...]
        ha = ((aa_ref[0] + aa_ref[1] + hsa_ref[...]) * dis
              + b_ref[:, :DP])
        hb = ((ab_ref[0] + ab_ref[1] + hsb_ref[...]) * dis
              + b_ref[:, DP:])
        h2 = jnp.maximum(jnp.concatenate([ha, hb], axis=1), 0.0)
        o_ref[...] = jnp.dot(h2, w_ref[...],
                             preferred_element_type=jnp.float32) * dis

    half = pl.BlockSpec((bm, DP), lambda i: (i, 0))
    return pl.pallas_call(
        body,
        grid=(n // bm,),
        in_specs=[
            pl.BlockSpec((NC, bm, DP), lambda i: (0, i, 0)),
            pl.BlockSpec((NC, bm, DP), lambda i: (0, i, 0)),
            half, half,
            pl.BlockSpec((bm, 1), lambda i: (i, 0)),
            pl.BlockSpec((1, d_hid), lambda i: (0, 0)),
            pl.BlockSpec((d_hid, d_out), lambda i: (0, 0)),
        ],
        out_specs=pl.BlockSpec((bm, d_out), lambda i: (i, 0)),
        out_shape=jax.ShapeDtypeStruct((n, d_out), jnp.float32),
    )


def _tc_final(n, bm, d_out):
    def body(a_ref, hs_ref, dis_ref, b_ref, o_ref):
        o_ref[...] = ((a_ref[0] + a_ref[1] + hs_ref[...])
                      * dis_ref[...] + b_ref[...])

    return pl.pallas_call(
        body,
        grid=(n // bm,),
        in_specs=[
            pl.BlockSpec((NC, bm, d_out), lambda i: (0, i, 0)),
            pl.BlockSpec((bm, d_out), lambda i: (i, 0)),
            pl.BlockSpec((bm, 1), lambda i: (i, 0)),
            pl.BlockSpec((1, d_out), lambda i: (0, 0)),
        ],
        out_specs=pl.BlockSpec((bm, d_out), lambda i: (i, 0)),
        out_shape=jax.ShapeDtypeStruct((n, d_out), jnp.float32),
    )


def kernel(x, edge_index, W1, b1, W2, b2):
    n, d_in = x.shape
    e = edge_index.shape[1]
    d_hid = W1.shape[1]
    d_out = W2.shape[1]

    nacc = -((n + 1) // -(NS * 8)) * NS * 8
    slab = nacc // NS

    ept = -(e // -(NW * 4 * CHUNK)) * 4 * CHUNK
    pad = NW * ept - e
    src = jnp.concatenate(
        [edge_index[0], jnp.zeros((pad,), jnp.int32)]) if pad else edge_index[0]
    dst = jnp.concatenate(
        [edge_index[1], jnp.full((pad,), nacc - 1, jnp.int32)]) if pad else edge_index[1]
    src3 = src.reshape(NW, ept // CHUNK, CHUNK)
    dst3 = dst.reshape(NW, ept // CHUNK, CHUNK)
    nchunk = ept // CHUNK

    ones_c = jnp.ones((CHUNK,), jnp.float32)
    zer1 = jnp.zeros((slab,), jnp.float32)
    zer_p = jnp.zeros((slab, DP), jnp.float32)

    degp = _sc_degree(nchunk, nacc)(dst3, ones_c, zer1).reshape(NC, nacc)
    deg = degp[0, :n] + degp[1, :n] + 1.0
    dis = lax.rsqrt(deg).reshape(n, 1)

    bm = 400
    h1 = _tc_matmul(n, bm, d_in, d_hid)(x, W1)
    hs1a, hs1b = _tc_scale_split(n, bm, d_hid)(h1, dis)
    acc1a, acc1b = _sc_messages(nchunk, nacc, n, 2)(
        src3, dst3, hs1a, hs1b, zer_p)
    hs2 = _tc_layer2(n, bm, d_hid, d_out)(
        acc1a, acc1b, hs1a, hs1b, dis, b1.reshape(1, d_hid), W2)
    acc2, = _sc_messages(nchunk, nacc, n, 1)(src3, dst3, hs2, zer_p)
    out = _tc_final(n, bm, d_out)(
        acc2, hs2, dis, b2.reshape(1, d_out))
    return out


# speedup vs baseline: 1.0025x; 1.0025x over previous
import jax
import jax.numpy as jnp
from jax import lax
from jax.experimental import pallas as pl
from jax.experimental.pallas import tpu as pltpu
from jax.experimental.pallas import tpu_sc as plsc

NC = 2
NS = 16
NW = NC * NS
CHUNK = 100
DP = 64


def _sc_degree(nchunk, nacc):
    slab = nacc // NS

    def body(dst_hbm, ones_hbm, zer_hbm, out_hbm, didx, ones_v, acc):
        cid = lax.axis_index("c")
        sid = lax.axis_index("s")
        wid = cid * NS + sid
        pltpu.sync_copy(zer_hbm, acc.at[pl.ds(sid * slab, slab)])
        pltpu.sync_copy(dst_hbm.at[wid], didx)
        pltpu.sync_copy(ones_hbm, ones_v)
        plsc.subcore_barrier()

        def step(j, carry):
            pltpu.sync_copy(ones_v, acc.at[didx.at[j]], add=True)
            return carry

        lax.fori_loop(0, nchunk, step, 0)
        plsc.subcore_barrier()
        pltpu.sync_copy(acc.at[pl.ds(sid * slab, slab)],
                        out_hbm.at[pl.ds(cid * nacc + sid * slab, slab)])

    return pl.kernel(
        body,
        out_type=jax.ShapeDtypeStruct((NC * nacc,), jnp.float32),
        mesh=plsc.VectorSubcoreMesh(core_axis_name="c", subcore_axis_name="s"),
        compiler_params=pltpu.CompilerParams(use_tc_tiling_on_sc=False),
        scratch_types=[
            pltpu.VMEM((nchunk, CHUNK), jnp.int32),
            pltpu.VMEM((CHUNK,), jnp.float32),
            pltpu.VMEM_SHARED((nacc,), jnp.float32),
        ],
    )


def _sc_messages(nchunk, nacc, n, npass):
    slab = nacc // NS
    stripe = n // NS

    def body(src_hbm, dst_hbm, *refs):
        tabs_hbm = refs[:npass]
        zer_hbm = refs[npass]
        outs_hbm = refs[npass + 1:2 * npass + 1]
        rest = refs[2 * npass + 1:]
        sidx, didx = rest[0], rest[1]
        rows = rest[2:6]
        tab, acc = rest[6], rest[7]
        gsem = rest[8:12]
        ssem = rest[12:16]
        cid = lax.axis_index("c")
        sid = lax.axis_index("s")
        wid = cid * NS + sid
        pltpu.sync_copy(src_hbm.at[wid], sidx)
        pltpu.sync_copy(dst_hbm.at[wid], didx)

        for p in range(npass):
            pltpu.sync_copy(zer_hbm, acc.at[pl.ds(sid * slab, slab)])
            pltpu.sync_copy(tabs_hbm[p].at[pl.ds(sid * stripe, stripe)],
                            tab.at[pl.ds(sid * stripe, stripe)])
            plsc.subcore_barrier()

            for b in range(3):
                pltpu.async_copy(tab.at[sidx.at[b]], rows[b], gsem[b])

            def phase(j, b, first):
                pltpu.make_async_copy(tab.at[sidx.at[j]], rows[b],
                                      gsem[b]).wait()
                pltpu.async_copy(rows[b], acc.at[didx.at[j]], ssem[b],
                                 add=True)
                nb = (b + 3) % 4
                if first:
                    pltpu.async_copy(tab.at[sidx.at[j + 3]], rows[nb],
                                     gsem[nb])
                else:
                    @pl.when(j + 3 < nchunk)
                    def _():
                        pltpu.make_async_copy(rows[nb],
                                              acc.at[didx.at[j - 1]],
                                              ssem[nb]).wait()
                        pltpu.async_copy(tab.at[sidx.at[j + 3]], rows[nb],
                                         gsem[nb])

            phase(0, 0, True)
            phase(1, 1, False)
            phase(2, 2, False)
            phase(3, 3, False)

            def step(i, carry):
                for b in range(4):
                    phase(4 * i + b, b, False)
                return carry

            lax.fori_loop(1, nchunk // 4, step, 0)
            for b in range(4):
                pltpu.make_async_copy(rows[b], acc.at[didx.at[0]],
                                      ssem[b]).wait()
            plsc.subcore_barrier()
            pltpu.sync_copy(acc.at[pl.ds(sid * slab, slab)],
                            outs_hbm[p].at[cid, pl.ds(sid * slab, slab)])
            if p + 1 < npass:
                plsc.subcore_barrier()

    return pl.kernel(
        body,
        out_type=[jax.ShapeDtypeStruct((NC, nacc, DP), jnp.float32)
                  for _ in range(npass)],
        mesh=plsc.VectorSubcoreMesh(core_axis_name="c", subcore_axis_name="s"),
        compiler_params=pltpu.CompilerParams(use_tc_tiling_on_sc=False),
        scratch_types=[
            pltpu.VMEM((nchunk, CHUNK), jnp.int32),
            pltpu.VMEM((nchunk, CHUNK), jnp.int32),
            pltpu.VMEM((CHUNK, DP), jnp.float32),
            pltpu.VMEM((CHUNK, DP), jnp.float32),
            pltpu.VMEM((CHUNK, DP), jnp.float32),
            pltpu.VMEM((CHUNK, DP), jnp.float32),
            pltpu.VMEM_SHARED((nacc, DP), jnp.float32),
            pltpu.VMEM_SHARED((nacc, DP), jnp.float32),
            pltpu.SemaphoreType.DMA,
            pltpu.SemaphoreType.DMA,
            pltpu.SemaphoreType.DMA,
            pltpu.SemaphoreType.DMA,
            pltpu.SemaphoreType.DMA,
            pltpu.SemaphoreType.DMA,
            pltpu.SemaphoreType.DMA,
            pltpu.SemaphoreType.DMA,
        ],
    )


def _tc_matmul(n, bm, d_in, d_out):
    def body(x_ref, w_ref, o_ref):
        o_ref[...] = jnp.dot(x_ref[...], w_ref[...],
                             preferred_element_type=jnp.float32)

    return pl.pallas_call(
        body,
        grid=(n // bm,),
        in_specs=[
            pl.BlockSpec((bm, d_in), lambda i: (i, 0)),
            pl.BlockSpec((d_in, d_out), lambda i: (0, 0)),
        ],
        out_specs=pl.BlockSpec((bm, d_out), lambda i: (i, 0)),
        out_shape=jax.ShapeDtypeStruct((n, d_out), jnp.float32),
    )


def kernel(x, edge_index, W1, b1, W2, b2):
    n, d_in = x.shape
    e = edge_index.shape[1]
    d_hid = W1.shape[1]
    d_out = W2.shape[1]

    nacc = -((n + 1) // -(NS * 8)) * NS * 8
    slab = nacc // NS

    ept = -(e // -(NW * 4 * CHUNK)) * 4 * CHUNK
    pad = NW * ept - e
    src = jnp.concatenate(
        [edge_index[0], jnp.zeros((pad,), jnp.int32)]) if pad else edge_index[0]
    dst = jnp.concatenate(
        [edge_index[1], jnp.full((pad,), nacc - 1, jnp.int32)]) if pad else edge_index[1]
    src3 = src.reshape(NW, ept // CHUNK, CHUNK)
    dst3 = dst.reshape(NW, ept // CHUNK, CHUNK)
    nchunk = ept // CHUNK

    ones_c = jnp.ones((CHUNK,), jnp.float32)
    zer1 = jnp.zeros((slab,), jnp.float32)
    zer_p = jnp.zeros((slab, DP), jnp.float32)

    degp = _sc_degree(nchunk, nacc)(dst3, ones_c, zer1).reshape(NC, nacc)
    deg = degp[0, :n] + degp[1, :n] + 1.0
    dis = lax.rsqrt(deg).reshape(n, 1)

    bm = 400
    h1 = _tc_matmul(n, bm, d_in, d_hid)(x, W1)
    hs1 = h1 * dis
    hs1a, hs1b = hs1[:, :DP], hs1[:, DP:]
    acc1a, acc1b = _sc_messages(nchunk, nacc, n, 2)(
        src3, dst3, hs1a, hs1b, zer_p)
    pre = jnp.concatenate(
        [acc1a[0, :n] + acc1a[1, :n] + hs1a,
         acc1b[0, :n] + acc1b[1, :n] + hs1b], axis=1)
    h2 = jnp.maximum(pre * dis + b1, 0.0)
    h3 = _tc_matmul(n, bm, d_hid, d_out)(h2, W2)
    hs2 = h3 * dis
    acc2, = _sc_messages(nchunk, nacc, n, 1)(src3, dst3, hs2, zer_p)
    out = (acc2[0, :n] + acc2[1, :n] + hs2) * dis + b2
    return out


# speedup vs baseline: 1.0143x; 1.0118x over previous
import jax
import jax.numpy as jnp
from jax import lax
from jax.experimental import pallas as pl
from jax.experimental.pallas import tpu as pltpu
from jax.experimental.pallas import tpu_sc as plsc

NC = 2
NS = 16
NW = NC * NS
CHUNK = 100
DP = 64


def _sc_degree(nchunk, nacc):
    slab = nacc // NS

    def body(dst_hbm, ones_hbm, zer_hbm, out_hbm, didx, ones_v, acc):
        cid = lax.axis_index("c")
        sid = lax.axis_index("s")
        wid = cid * NS + sid
        pltpu.sync_copy(zer_hbm, acc.at[pl.ds(sid * slab, slab)])
        pltpu.sync_copy(dst_hbm.at[wid], didx)
        pltpu.sync_copy(ones_hbm, ones_v)
        plsc.subcore_barrier()

        def step(j, carry):
            pltpu.sync_copy(ones_v, acc.at[didx.at[j]], add=True)
            return carry

        lax.fori_loop(0, nchunk, step, 0)
        plsc.subcore_barrier()
        pltpu.sync_copy(acc.at[pl.ds(sid * slab, slab)],
                        out_hbm.at[pl.ds(cid * nacc + sid * slab, slab)])

    return pl.kernel(
        body,
        out_type=jax.ShapeDtypeStruct((NC * nacc,), jnp.float32),
        mesh=plsc.VectorSubcoreMesh(core_axis_name="c", subcore_axis_name="s"),
        compiler_params=pltpu.CompilerParams(use_tc_tiling_on_sc=False),
        scratch_types=[
            pltpu.VMEM((nchunk, CHUNK), jnp.int32),
            pltpu.VMEM((CHUNK,), jnp.float32),
            pltpu.VMEM_SHARED((nacc,), jnp.float32),
        ],
    )


def _sc_messages(nchunk, nacc, n, npass):
    slab = nacc // NS
    stripe = n // NS

    def body(src_hbm, dst_hbm, *refs):
        tabs_hbm = refs[:npass]
        zer_hbm = refs[npass]
        outs_hbm = refs[npass + 1:2 * npass + 1]
        rest = refs[2 * npass + 1:]
        sidx, didx = rest[0], rest[1]
        rows = rest[2:6]
        tab, acc = rest[6], rest[7]
        gsem = rest[8:12]
        ssem = rest[12:16]
        cid = lax.axis_index("c")
        sid = lax.axis_index("s")
        wid = cid * NS + sid
        pltpu.sync_copy(src_hbm.at[wid], sidx)
        pltpu.sync_copy(dst_hbm.at[wid], didx)

        for p in range(npass):
            pltpu.sync_copy(zer_hbm, acc.at[pl.ds(sid * slab, slab)])
            pltpu.sync_copy(tabs_hbm[p].at[pl.ds(sid * stripe, stripe)],
                            tab.at[pl.ds(sid * stripe, stripe)])
            plsc.subcore_barrier()

            for b in range(3):
                pltpu.async_copy(tab.at[sidx.at[b]], rows[b], gsem[b])

            def phase(j, b, first):
                pltpu.make_async_copy(tab.at[sidx.at[j]], rows[b],
                                      gsem[b]).wait()
                pltpu.async_copy(rows[b], acc.at[didx.at[j]], ssem[b],
                                 add=True)
                nb = (b + 3) % 4
                if first:
                    pltpu.async_copy(tab.at[sidx.at[j + 3]], rows[nb],
                                     gsem[nb])
                else:
                    @pl.when(j + 3 < nchunk)
                    def _():
                        pltpu.make_async_copy(rows[nb],
                                              acc.at[didx.at[j - 1]],
                                              ssem[nb]).wait()
                        pltpu.async_copy(tab.at[sidx.at[j + 3]], rows[nb],
                                         gsem[nb])

            phase(0, 0, True)
            phase(1, 1, False)
            phase(2, 2, False)
            phase(3, 3, False)

            def step(i, carry):
                for b in range(4):
                    phase(4 * i + b, b, False)
                return carry

            lax.fori_loop(1, nchunk // 4, step, 0)
            for b in range(4):
                pltpu.make_async_copy(rows[b], acc.at[didx.at[0]],
                                      ssem[b]).wait()
            plsc.subcore_barrier()
            pltpu.sync_copy(acc.at[pl.ds(sid * slab, slab)],
                            outs_hbm[p].at[cid, pl.ds(sid * slab, slab)])
            if p + 1 < npass:
                plsc.subcore_barrier()

    return pl.kernel(
        body,
        out_type=[jax.ShapeDtypeStruct((NC, nacc, DP), jnp.float32)
                  for _ in range(npass)],
        mesh=plsc.VectorSubcoreMesh(core_axis_name="c", subcore_axis_name="s"),
        compiler_params=pltpu.CompilerParams(use_tc_tiling_on_sc=False),
        scratch_types=[
            pltpu.VMEM((nchunk, CHUNK), jnp.int32),
            pltpu.VMEM((nchunk, CHUNK), jnp.int32),
            pltpu.VMEM((CHUNK, DP), jnp.float32),
            pltpu.VMEM((CHUNK, DP), jnp.float32),
            pltpu.VMEM((CHUNK, DP), jnp.float32),
            pltpu.VMEM((CHUNK, DP), jnp.float32),
            pltpu.VMEM_SHARED((nacc, DP), jnp.float32),
            pltpu.VMEM_SHARED((nacc, DP), jnp.float32),
            pltpu.SemaphoreType.DMA,
            pltpu.SemaphoreType.DMA,
            pltpu.SemaphoreType.DMA,
            pltpu.SemaphoreType.DMA,
            pltpu.SemaphoreType.DMA,
            pltpu.SemaphoreType.DMA,
            pltpu.SemaphoreType.DMA,
            pltpu.SemaphoreType.DMA,
        ],
    )


def _tc_mm_scale(n, bm, d_in, d_hid):
    def body(x_ref, w_ref, dis_ref, oa_ref, ob_ref):
        h = jnp.dot(x_ref[...], w_ref[...], preferred_element_type=jnp.float32)
        hs = h * dis_ref[...]
        oa_ref[...] = hs[:, :DP]
        ob_ref[...] = hs[:, DP:]

    return pl.pallas_call(
        body,
        grid=(n // bm,),
        in_specs=[
            pl.BlockSpec((bm, d_in), lambda i: (i, 0)),
            pl.BlockSpec((d_in, d_hid), lambda i: (0, 0)),
            pl.BlockSpec((bm, 1), lambda i: (i, 0)),
        ],
        out_specs=[
            pl.BlockSpec((bm, DP), lambda i: (i, 0)),
            pl.BlockSpec((bm, DP), lambda i: (i, 0)),
        ],
        out_shape=[
            jax.ShapeDtypeStruct((n, DP), jnp.float32),
            jax.ShapeDtypeStruct((n, DP), jnp.float32),
        ],
    )


def _tc_layer2(n, bm, d_hid, d_out):
    def body(aa_ref, ab_ref, hsa_ref, hsb_ref, dis_ref, b_ref, w_ref, o_ref):
        dis = dis_ref[...]
        ha = ((aa_ref[0] + aa_ref[1] + hsa_ref[...]) * dis
              + b_ref[:, :DP])
        hb = ((ab_ref[0] + ab_ref[1] + hsb_ref[...]) * dis
              + b_ref[:, DP:])
        h2 = jnp.maximum(jnp.concatenate([ha, hb], axis=1), 0.0)
        o_ref[...] = jnp.dot(h2, w_ref[...],
                             preferred_element_type=jnp.float32) * dis

    half = pl.BlockSpec((bm, DP), lambda i: (i, 0))
    return pl.pallas_call(
        body,
        grid=(n // bm,),
        in_specs=[
            pl.BlockSpec((NC, bm, DP), lambda i: (0, i, 0)),
            pl.BlockSpec((NC, bm, DP), lambda i: (0, i, 0)),
            half, half,
            pl.BlockSpec((bm, 1), lambda i: (i, 0)),
            pl.BlockSpec((1, d_hid), lambda i: (0, 0)),
            pl.BlockSpec((d_hid, d_out), lambda i: (0, 0)),
        ],
        out_specs=pl.BlockSpec((bm, d_out), lambda i: (i, 0)),
        out_shape=jax.ShapeDtypeStruct((n, d_out), jnp.float32),
    )


def _tc_final(n, bm, d_out):
    def body(a_ref, hs_ref, dis_ref, b_ref, o_ref):
        o_ref[...] = ((a_ref[0] + a_ref[1] + hs_ref[...])
                      * dis_ref[...] + b_ref[...])

    return pl.pallas_call(
        body,
        grid=(n // bm,),
        in_specs=[
            pl.BlockSpec((NC, bm, d_out), lambda i: (0, i, 0)),
            pl.BlockSpec((bm, d_out), lambda i: (i, 0)),
            pl.BlockSpec((bm, 1), lambda i: (i, 0)),
            pl.BlockSpec((1, d_out), lambda i: (0, 0)),
        ],
        out_specs=pl.BlockSpec((bm, d_out), lambda i: (i, 0)),
        out_shape=jax.ShapeDtypeStruct((n, d_out), jnp.float32),
    )


def kernel(x, edge_index, W1, b1, W2, b2):
    n, d_in = x.shape
    e = edge_index.shape[1]
    d_hid = W1.shape[1]
    d_out = W2.shape[1]

    nacc = -((n + 1) // -(NS * 8)) * NS * 8
    slab = nacc // NS

    ept = -(e // -(NW * 4 * CHUNK)) * 4 * CHUNK
    pad = NW * ept - e
    src = jnp.concatenate(
        [edge_index[0], jnp.zeros((pad,), jnp.int32)]) if pad else edge_index[0]
    dst = jnp.concatenate(
        [edge_index[1], jnp.full((pad,), nacc - 1, jnp.int32)]) if pad else edge_index[1]
    src3 = src.reshape(NW, ept // CHUNK, CHUNK)
    dst3 = dst.reshape(NW, ept // CHUNK, CHUNK)
    nchunk = ept // CHUNK

    ones_c = jnp.ones((CHUNK,), jnp.float32)
    zer1 = jnp.zeros((slab,), jnp.float32)
    zer_p = jnp.zeros((slab, DP), jnp.float32)

    degp = _sc_degree(nchunk, nacc)(dst3, ones_c, zer1).reshape(NC, nacc)
    deg = degp[0, :n] + degp[1, :n] + 1.0
    dis = lax.rsqrt(deg).reshape(n, 1)

    bm = 400
    hs1a, hs1b = _tc_mm_scale(n, bm, d_in, d_hid)(x, W1, dis)
    acc1a, acc1b = _sc_messages(nchunk, nacc, n, 2)(
        src3, dst3, hs1a, hs1b, zer_p)
    hs2 = _tc_layer2(n, bm, d_hid, d_out)(
        acc1a, acc1b, hs1a, hs1b, dis, b1.reshape(1, d_hid), W2)
    acc2, = _sc_messages(nchunk, nacc, n, 1)(src3, dst3, hs2, zer_p)
    out = _tc_final(n, bm, d_out)(
        acc2, hs2, dis, b2.reshape(1, d_out))
    return out
